# baseline (speedup 1.0000x reference)
import functools

import jax
import jax.numpy as jnp
from jax import lax
from jax.experimental import pallas as pl
from jax.experimental.pallas import tpu as pltpu
from jax.experimental.pallas import tpu_sc as plsc

NUM_FACTORS = 32
BATCH = 16384
NC = 2
NS = 16
NW = NC * NS
B_PER_W = BATCH // NW
CHUNK = 128
NCHUNK = B_PER_W // CHUNK
GROUPS = B_PER_W // 16


def _cf_kernel(user_hbm, item_hbm, uf_hbm, if_hbm, p_hbm, out_hbm,
               uidx_v, iidx_v, urows_v, irows_v, p_v, outb_v, sem):
    wid = lax.axis_index("s") * NC + lax.axis_index("c")
    row0 = wid * NCHUNK

    pltpu.sync_copy(user_hbm.at[pl.ds(row0, NCHUNK)], uidx_v)
    pltpu.sync_copy(item_hbm.at[pl.ds(row0, NCHUNK)], iidx_v)
    pltpu.sync_copy(p_hbm, p_v)

    copies = []
    for j in range(NCHUNK):
        copies.append(pltpu.async_copy(
            uf_hbm.at[uidx_v.at[j]], urows_v.at[pl.ds(j * CHUNK, CHUNK)], sem))
        copies.append(pltpu.async_copy(
            if_hbm.at[iidx_v.at[j]], irows_v.at[pl.ds(j * CHUNK, CHUNK)], sem))
    for c in copies:
        c.wait()

    w_lo = p_v[pl.ds(0, 16)]
    w_hi = p_v[pl.ds(16, 16)]
    tail = p_v[pl.ds(32, 16)]
    ws = [w_lo[f] for f in range(16)] + [w_hi[f] for f in range(16)]
    bias = tail[0]
    lanes = lax.iota(jnp.int32, 16)

    def body(g, carry):
        rows = g * 16 + lanes
        acc = jnp.full((16,), bias, dtype=jnp.float32)
        for f in range(NUM_FACTORS):
            col = jnp.full((16,), f, dtype=jnp.int32)
            uv = plsc.load_gather(urows_v, [rows, col])
            iv = plsc.load_gather(irows_v, [rows, col])
            acc = acc + (uv * iv) * ws[f]
        outb_v[pl.ds(g * 16, 16)] = acc
        return carry

    lax.fori_loop(0, GROUPS, body, 0)

    pltpu.sync_copy(outb_v, out_hbm.at[pl.ds(wid * B_PER_W, B_PER_W)])


@jax.jit
def _cf_call(user, item, user_factors, item_factors, fc_w, fc_b):
    mesh = plsc.VectorSubcoreMesh(core_axis_name="c", subcore_axis_name="s")
    k = functools.partial(
        pl.kernel,
        out_type=jax.ShapeDtypeStruct((BATCH,), jnp.float32),
        mesh=mesh,
        scratch_types=[
            pltpu.VMEM((NCHUNK, CHUNK), jnp.int32),
            pltpu.VMEM((NCHUNK, CHUNK), jnp.int32),
            pltpu.VMEM((B_PER_W, NUM_FACTORS), jnp.float32),
            pltpu.VMEM((B_PER_W, NUM_FACTORS), jnp.float32),
            pltpu.VMEM((48,), jnp.float32),
            pltpu.VMEM((B_PER_W,), jnp.float32),
            pltpu.SemaphoreType.DMA,
        ],
        compiler_params=pltpu.CompilerParams(
            needs_layout_passes=False, use_tc_tiling_on_sc=False),
    )(_cf_kernel)
    user2d = user.reshape(NW * NCHUNK, CHUNK)
    item2d = item.reshape(NW * NCHUNK, CHUNK)
    params = jnp.concatenate(
        [fc_w.reshape(NUM_FACTORS), fc_b, jnp.zeros((15,), jnp.float32)])
    return k(user2d, item2d, user_factors, item_factors, params)


def kernel(user, item, user_factors, item_factors, fc_w, fc_b):
    out = _cf_call(user.astype(jnp.int32), item.astype(jnp.int32),
                   user_factors, item_factors, fc_w, fc_b)
    return out.reshape(BATCH, 1)
